# Initial kernel scaffold; baseline (speedup 1.0000x reference)
import jax
import jax.numpy as jnp
from jax.experimental import pallas as pl


def kernel(input_ids, offsets, table):
    raise NotImplementedError("write your pallas kernel here")


# speedup vs baseline: 7.8391x; 7.8391x over previous
import functools

import jax
import jax.numpy as jnp
from jax import lax
from jax.experimental import pallas as pl
from jax.experimental.pallas import tpu as pltpu
from jax.experimental.pallas import tpu_sc as plsc

B, NQ, T = 16, 8, 2048
DIM = 128
N = B * NQ * T
NW = 32
PER_W = N // NW
IDX_ROWS = PER_W // 128
C = 256
NCHUNK = PER_W // C
STREAMS = C // 128


def _body(idx_hbm, offs_hbm, table_hbm, out_hbm,
          offs_v, idx_v, rows_v, gs0, gs1, os0, os1):
    cid = lax.axis_index("c")
    sid = lax.axis_index("s")
    wid = sid * 2 + cid
    base = wid * PER_W
    row_base = wid * IDX_ROWS

    gsems = (gs0, gs1)
    osems = (os0, os1)

    pltpu.sync_copy(offs_hbm, offs_v)
    pltpu.sync_copy(idx_hbm.at[pl.ds(row_base, IDX_ROWS)], idx_v)

    def add_body(i, carry):
        fr = (base + i * 128) // T
        q = lax.rem(fr, NQ)
        off = offs_v[pl.ds(q * 16, 16)]
        for j in range(8):
            sl = pl.ds(j * 16, 16)
            idx_v[i, sl] = idx_v[i, sl] + off
        return carry

    lax.fori_loop(0, IDX_ROWS, add_body, 0)

    def fire_gather(g, b):
        r0 = g * STREAMS
        for s in range(STREAMS):
            pltpu.async_copy(
                table_hbm.at[idx_v.at[r0 + s]],
                rows_v.at[b, pl.ds(s * 128, 128)],
                gsems[b],
            )

    def wait_gather(b):
        pltpu.make_async_copy(
            table_hbm.at[idx_v.at[0]], rows_v.at[b], gsems[b]
        ).wait()

    def fire_store(g, b):
        pltpu.async_copy(
            rows_v.at[b], out_hbm.at[pl.ds(base + g * C, C)], osems[b]
        )

    def wait_store(b):
        pltpu.make_async_copy(
            rows_v.at[b], out_hbm.at[pl.ds(base, C)], osems[b]
        ).wait()

    fire_gather(0, 0)
    wait_gather(0)
    fire_store(0, 0)
    fire_gather(1, 1)
    wait_gather(1)
    fire_store(1, 1)

    def loop_body(g2, carry):
        for b in range(2):
            g = g2 * 2 + b
            wait_store(b)
            fire_gather(g, b)
            wait_gather(b)
            fire_store(g, b)
        return carry

    lax.fori_loop(1, NCHUNK // 2, loop_body, 0)
    wait_store(0)
    wait_store(1)


@jax.jit
def _gather(idx2d, offs16, table):
    mesh = plsc.VectorSubcoreMesh(core_axis_name="c", subcore_axis_name="s")
    f = functools.partial(
        pl.kernel,
        out_type=jax.ShapeDtypeStruct((N, DIM), jnp.float32),
        mesh=mesh,
        scratch_types=[
            pltpu.VMEM((NQ * 16,), jnp.int32),
            pltpu.VMEM((IDX_ROWS, 128), jnp.int32),
            pltpu.VMEM((2, C, DIM), jnp.float32),
            pltpu.SemaphoreType.DMA,
            pltpu.SemaphoreType.DMA,
            pltpu.SemaphoreType.DMA,
            pltpu.SemaphoreType.DMA,
        ],
    )(_body)
    return f(idx2d, offs16, table)


def kernel(input_ids, offsets, table):
    idx2d = input_ids.reshape(N // 128, 128)
    offs_rep = jnp.repeat(offsets, 16)
    out = _gather(idx2d, offs_rep, table)
    return out.reshape(B, NQ, T, DIM)


# speedup vs baseline: 8.1772x; 1.0431x over previous
import functools

import jax
import jax.numpy as jnp
from jax import lax
from jax.experimental import pallas as pl
from jax.experimental.pallas import tpu as pltpu
from jax.experimental.pallas import tpu_sc as plsc

B, NQ, T = 16, 8, 2048
DIM = 128
N = B * NQ * T
NW = 32
PER_W = N // NW
C = 128
NCHUNK = PER_W // C
NBUF = 4
ROUNDS = NCHUNK // NBUF


def _body(idx_hbm, offs_hbm, table_hbm, out_hbm, offs_v, idx_v, rows_v,
          gs0, gs1, gs2, gs3, os0, os1, os2, os3):
    cid = lax.axis_index("c")
    sid = lax.axis_index("s")
    wid = sid * 2 + cid
    base = wid * PER_W

    gsems = (gs0, gs1, gs2, gs3)
    osems = (os0, os1, os2, os3)

    pltpu.sync_copy(offs_hbm, offs_v)
    pltpu.sync_copy(idx_hbm.at[pl.ds(wid * NCHUNK, NCHUNK)], idx_v)

    def prep(g):
        q = lax.rem((base + g * C) // T, NQ)
        off = offs_v[pl.ds(q * 16, 16)]
        for j in range(8):
            sl = pl.ds(j * 16, 16)
            idx_v[g, sl] = idx_v[g, sl] + off

    def fire_gather(g, b):
        pltpu.async_copy(table_hbm.at[idx_v.at[g]], rows_v.at[b], gsems[b])

    def wait_gather(b):
        pltpu.make_async_copy(
            table_hbm.at[idx_v.at[0]], rows_v.at[b], gsems[b]
        ).wait()

    def fire_store(g, b):
        pltpu.async_copy(
            rows_v.at[b], out_hbm.at[pl.ds(base + g * C, C)], osems[b]
        )

    def wait_store(b):
        pltpu.make_async_copy(
            rows_v.at[b], out_hbm.at[pl.ds(base, C)], osems[b]
        ).wait()

    def step(g, b, fire_next, wait_next_store):
        gn = g + NBUF - 1
        bn = (b + NBUF - 1) % NBUF
        if fire_next:
            prep(gn)
            if wait_next_store:
                wait_store(bn)
            fire_gather(gn, bn)
        wait_gather(b)
        fire_store(g, b)

    for g in range(NBUF - 1):
        prep(g)
        fire_gather(g, g)
    step(0, 0, True, False)
    for b in range(1, NBUF):
        step(b, b, True, True)

    def round_body(r, carry):
        for b in range(NBUF):
            step(r * NBUF + b, b, True, True)
        return carry

    lax.fori_loop(1, ROUNDS - 1, round_body, 0)

    g0 = (ROUNDS - 1) * NBUF
    step(g0, 0, True, True)
    for b in range(1, NBUF):
        step(g0 + b, b, False, False)

    for b in range(NBUF):
        wait_store(b)


@jax.jit
def _gather(idx2d, offs_rep, table):
    mesh = plsc.VectorSubcoreMesh(core_axis_name="c", subcore_axis_name="s")
    f = functools.partial(
        pl.kernel,
        out_type=jax.ShapeDtypeStruct((N, DIM), jnp.float32),
        mesh=mesh,
        scratch_types=[
            pltpu.VMEM((NQ * 16,), jnp.int32),
            pltpu.VMEM((NCHUNK, C), jnp.int32),
            pltpu.VMEM((NBUF, C, DIM), jnp.float32),
            pltpu.SemaphoreType.DMA,
            pltpu.SemaphoreType.DMA,
            pltpu.SemaphoreType.DMA,
            pltpu.SemaphoreType.DMA,
            pltpu.SemaphoreType.DMA,
            pltpu.SemaphoreType.DMA,
            pltpu.SemaphoreType.DMA,
            pltpu.SemaphoreType.DMA,
        ],
    )(_body)
    return f(idx2d, offs_rep, table)


def kernel(input_ids, offsets, table):
    idx2d = input_ids.reshape(N // C, C)
    offs_rep = jnp.repeat(offsets, 16)
    out = _gather(idx2d, offs_rep, table)
    return out.reshape(B, NQ, T, DIM)


# speedup vs baseline: 8.1890x; 1.0014x over previous
import functools

import jax
import jax.numpy as jnp
from jax import lax
from jax.experimental import pallas as pl
from jax.experimental.pallas import tpu as pltpu
from jax.experimental.pallas import tpu_sc as plsc

B, NQ, T = 16, 8, 2048
DIM = 128
N = B * NQ * T
NW = 32
PER_W = N // NW
C = 128
NCHUNK = PER_W // C
NBUF = 6
PRO = 4


def _body(idx_hbm, offs_hbm, table_hbm, out_hbm, offs_v, idx_v, rows_v,
          gs0, gs1, gs2, gs3, gs4, gs5, os0, os1, os2, os3, os4, os5):
    cid = lax.axis_index("c")
    sid = lax.axis_index("s")
    wid = sid * 2 + cid
    base = wid * PER_W

    gsems = (gs0, gs1, gs2, gs3, gs4, gs5)
    osems = (os0, os1, os2, os3, os4, os5)

    pltpu.sync_copy(offs_hbm, offs_v)
    pltpu.sync_copy(idx_hbm.at[pl.ds(wid * NCHUNK, NCHUNK)], idx_v)

    def prep(g):
        q = lax.rem((base + g * C) // T, NQ)
        off = offs_v[pl.ds(q * 16, 16)]
        for j in range(8):
            sl = pl.ds(j * 16, 16)
            idx_v[g, sl] = idx_v[g, sl] + off

    def fire_gather(g, b):
        pltpu.async_copy(table_hbm.at[idx_v.at[g]], rows_v.at[b], gsems[b])

    def wait_gather(b):
        pltpu.make_async_copy(
            table_hbm.at[idx_v.at[0]], rows_v.at[b], gsems[b]
        ).wait()

    def fire_store(g, b):
        pltpu.async_copy(
            rows_v.at[b], out_hbm.at[pl.ds(base + g * C, C)], osems[b]
        )

    def wait_store(b):
        pltpu.make_async_copy(
            rows_v.at[b], out_hbm.at[pl.ds(base, C)], osems[b]
        ).wait()

    def step(g, b, fire, wait_st):
        if fire:
            gn = g + PRO
            bn = (b + PRO) % NBUF
            prep(gn)
            if wait_st:
                wait_store(bn)
            fire_gather(gn, bn)
        wait_gather(b)
        fire_store(g, b)

    for g in range(PRO):
        prep(g)
        fire_gather(g, g % NBUF)
    for g in range(NBUF - PRO):
        step(g, g % NBUF, True, False)
    for g in range(NBUF - PRO, NBUF):
        step(g, g % NBUF, True, True)

    def round_body(r, carry):
        for b in range(NBUF):
            step(r * NBUF + b, b, True, True)
        return carry

    lax.fori_loop(1, (NCHUNK - PRO) // NBUF, round_body, 0)

    for g in range(NCHUNK - PRO, NCHUNK):
        step(g, g % NBUF, False, False)

    for b in range(NBUF):
        wait_store(b)


@jax.jit
def _gather(idx2d, offs_rep, table):
    mesh = plsc.VectorSubcoreMesh(core_axis_name="c", subcore_axis_name="s")
    f = functools.partial(
        pl.kernel,
        out_type=jax.ShapeDtypeStruct((N, DIM), jnp.float32),
        mesh=mesh,
        scratch_types=[
            pltpu.VMEM((NQ * 16,), jnp.int32),
            pltpu.VMEM((NCHUNK, C), jnp.int32),
            pltpu.VMEM((NBUF, C, DIM), jnp.float32),
        ] + [pltpu.SemaphoreType.DMA] * (2 * NBUF),
    )(_body)
    return f(idx2d, offs_rep, table)


def kernel(input_ids, offsets, table):
    idx2d = input_ids.reshape(N // C, C)
    offs_rep = jnp.repeat(offsets, 16)
    out = _gather(idx2d, offs_rep, table)
    return out.reshape(B, NQ, T, DIM)


# speedup vs baseline: 8.2034x; 1.0018x over previous
import functools

import jax
import jax.numpy as jnp
from jax import lax
from jax.experimental import pallas as pl
from jax.experimental.pallas import tpu as pltpu
from jax.experimental.pallas import tpu_sc as plsc

B, NQ, T = 16, 8, 2048
DIM = 128
N = B * NQ * T
NW = 32
PER_W = N // NW
C = 128
NCHUNK = PER_W // C
NBUF = 7
PRO = 5


def _body(idx_hbm, offs_hbm, table_hbm, out_hbm, offs_v, idx_v, rows_v,
          *sems):
    cid = lax.axis_index("c")
    sid = lax.axis_index("s")
    wid = sid * 2 + cid
    base = wid * PER_W

    gsems = sems[:NBUF]
    osems = sems[NBUF:]

    pltpu.sync_copy(offs_hbm, offs_v)
    pltpu.sync_copy(idx_hbm.at[pl.ds(wid * NCHUNK, NCHUNK)], idx_v)

    def prep(g):
        q = lax.rem((base + g * C) // T, NQ)
        off = offs_v[pl.ds(q * 16, 16)]
        for j in range(8):
            sl = pl.ds(j * 16, 16)
            idx_v[g, sl] = idx_v[g, sl] + off

    def fire_gather(g, b):
        pltpu.async_copy(table_hbm.at[idx_v.at[g]], rows_v.at[b], gsems[b])

    def wait_gather(b):
        pltpu.make_async_copy(
            table_hbm.at[idx_v.at[0]], rows_v.at[b], gsems[b]
        ).wait()

    def fire_store(g, b):
        pltpu.async_copy(
            rows_v.at[b], out_hbm.at[pl.ds(base + g * C, C)], osems[b]
        )

    def wait_store(b):
        pltpu.make_async_copy(
            rows_v.at[b], out_hbm.at[pl.ds(base, C)], osems[b]
        ).wait()

    def step(g, b, fire, wait_st):
        if fire:
            gn = g + PRO
            bn = (b + PRO) % NBUF
            prep(gn)
            if wait_st:
                wait_store(bn)
            fire_gather(gn, bn)
        wait_gather(b)
        fire_store(g, b)

    for g in range(PRO):
        prep(g)
        fire_gather(g, g % NBUF)
    for g in range(NBUF):
        step(g, g % NBUF, g + PRO < NCHUNK, g >= NBUF - PRO)

    def round_body(r, carry):
        for b in range(NBUF):
            step(r * NBUF + b, b, True, True)
        return carry

    rounds = (NCHUNK - PRO) // NBUF
    lax.fori_loop(1, rounds, round_body, 0)

    for g in range(rounds * NBUF, NCHUNK):
        step(g, g % NBUF, g + PRO < NCHUNK, True)

    for b in range(NBUF):
        wait_store(b)


@jax.jit
def _gather(idx2d, offs_rep, table):
    mesh = plsc.VectorSubcoreMesh(core_axis_name="c", subcore_axis_name="s")
    f = functools.partial(
        pl.kernel,
        out_type=jax.ShapeDtypeStruct((N, DIM), jnp.float32),
        mesh=mesh,
        scratch_types=[
            pltpu.VMEM((NQ * 16,), jnp.int32),
            pltpu.VMEM((NCHUNK, C), jnp.int32),
            pltpu.VMEM((NBUF, C, DIM), jnp.float32),
        ] + [pltpu.SemaphoreType.DMA] * (2 * NBUF),
    )(_body)
    return f(idx2d, offs_rep, table)


def kernel(input_ids, offsets, table):
    idx2d = input_ids.reshape(N // C, C)
    offs_rep = jnp.repeat(offsets, 16)
    out = _gather(idx2d, offs_rep, table)
    return out.reshape(B, NQ, T, DIM)
